# baseline (speedup 1.0000x reference)
import functools
import jax
import jax.numpy as jnp
from jax import lax
from jax.experimental import pallas as pl
from jax.experimental.pallas import tpu as pltpu
from jax.experimental.pallas import tpu_sc as plsc

_N = 20000
_H = 512
_W = 512
_FX = 600.0
_FY = 600.0
_CX = 256.0
_CY = 256.0
_NEAR = 0.01
_FAR = 100.0
_MAXR = 8

_NC = 2
_NS = 16
_NPAD = 20480
_CHUNK = _NPAD // _NS
_PATCH = (2 * _MAXR + 1) ** 2
_KV = 19
_GRP = 8
_GLANES = _GRP * _KV * 16
_HHALF = _H // _NC
_PIXH = _HHALF * _W
_STRIPE = _PIXH // _NS
_NCHUNK = 2048


def _params_body(pos_ref, scl_ref, rot_ref, col_ref, opa_ref, out_ref):
    x = pos_ref[0:1, :]
    y = pos_ref[1:2, :]
    z = pos_ref[2:3, :]
    depths = -z
    qw = rot_ref[0:1, :]
    qx = rot_ref[1:2, :]
    qy = rot_ref[2:3, :]
    qz = rot_ref[3:4, :]
    qn = jnp.maximum(jnp.sqrt(qw * qw + qx * qx + qy * qy + qz * qz), 1e-12)
    qw = qw / qn
    qx = qx / qn
    qy = qy / qn
    qz = qz / qn
    r00 = 1 - 2 * (qy * qy + qz * qz)
    r01 = 2 * (qx * qy - qw * qz)
    r02 = 2 * (qx * qz + qw * qy)
    r10 = 2 * (qx * qy + qw * qz)
    r11 = 1 - 2 * (qx * qx + qz * qz)
    r12 = 2 * (qy * qz - qw * qx)
    r20 = 2 * (qx * qz - qw * qy)
    r21 = 2 * (qy * qz + qw * qx)
    r22 = 1 - 2 * (qx * qx + qy * qy)
    s0 = scl_ref[0:1, :]
    s1 = scl_ref[1:2, :]
    s2 = scl_ref[2:3, :]
    v0 = s0 * s0
    v1 = s1 * s1
    v2 = s2 * s2
    c00 = r00 * r00 * v0 + r01 * r01 * v1 + r02 * r02 * v2
    c01 = r00 * r10 * v0 + r01 * r11 * v1 + r02 * r12 * v2
    c02 = r00 * r20 * v0 + r01 * r21 * v1 + r02 * r22 * v2
    c11 = r10 * r10 * v0 + r11 * r11 * v1 + r12 * r12 * v2
    c12 = r10 * r20 * v0 + r11 * r21 * v1 + r12 * r22 * v2
    c22 = r20 * r20 * v0 + r21 * r21 * v1 + r22 * r22 * v2
    z_safe = jnp.maximum(jnp.abs(z), 0.01) * jnp.sign(z + 1e-8)
    z2 = z_safe * z_safe
    j00 = _FX / -z_safe
    j02 = _FX * x / z2
    j11 = _FY / z_safe
    j12 = _FY * y / z2
    a = j00 * j00 * c00 + 2.0 * j00 * j02 * c02 + j02 * j02 * c22
    b = j00 * j11 * c01 + j00 * j12 * c02 + j02 * j11 * c12 + j02 * j12 * c22
    d = j11 * j11 * c11 + 2.0 * j11 * j12 * c12 + j12 * j12 * c22
    u = _FX * x / -z_safe + _CX
    v = _FY * -y / -z_safe + _CY
    trace = a + d
    det = jnp.maximum(a * d - b * b, 1e-6)
    disc = jnp.maximum(trace * trace - 4.0 * det, 0.0)
    lam = (trace + jnp.sqrt(disc)) * 0.5
    radii = jnp.minimum(3.0 * jnp.sqrt(jnp.maximum(lam, 1e-6)), float(_MAXR))
    vis = (depths > _NEAR) & (depths < _FAR)
    vis = vis & (u + radii > 0) & (u - radii < _W)
    vis = vis & (v + radii > 0) & (v - radii < _H)
    opac = opa_ref[0:1, :]
    vis = vis & (opac >= 1.0 / 255.0)
    aa = a + 0.3
    dd = d + 0.3
    det2 = jnp.maximum(aa * dd - b * b, 1e-6)
    inv_a = dd / det2
    inv_b = -b / det2
    inv_d = aa / det2
    cxpf = jnp.round(u)
    cypf = jnp.round(v)
    half0 = vis & (cypf + _MAXR >= 0) & (cypf - _MAXR <= _HHALF - 1)
    half1 = vis & (cypf + _MAXR >= _HHALF) & (cypf - _MAXR <= _H - 1)
    out_ref[0:1, :] = u
    out_ref[1:2, :] = v
    out_ref[2:3, :] = inv_a
    out_ref[3:4, :] = inv_b
    out_ref[4:5, :] = inv_d
    out_ref[5:6, :] = radii * radii
    out_ref[6:7, :] = opac
    out_ref[7:8, :] = col_ref[0:1, :]
    out_ref[8:9, :] = col_ref[1:2, :]
    out_ref[9:10, :] = col_ref[2:3, :]
    out_ref[10:11, :] = half0.astype(jnp.float32)
    out_ref[11:12, :] = cxpf
    out_ref[12:13, :] = cypf
    out_ref[13:14, :] = jnp.zeros_like(u)
    out_ref[14:15, :] = jnp.zeros_like(u)
    out_ref[15:16, :] = half1.astype(jnp.float32)


def _patch_consts(k):
    o = lax.iota(jnp.int32, 16) + (k * 16)
    dx = (o % 17 - _MAXR).astype(jnp.float32)
    dy = (o // 17 - _MAXR).astype(jnp.float32)
    valid = o < _PATCH
    return dx, dy, valid


def _sc_splat_body(prm_hbm, flg_hbm, zer_hbm, out_hbm,
                   flg_v, gidx_v, g16_v, prmg_v, vals_v, idx_v,
                   n0_v, n1_v, n2_v, n3_v,
                   pr_s, pg_s, pb_s, pa_s, sem0, sem1, gsem):
    c = lax.axis_index("c")
    s = lax.axis_index("s")
    planes = (pr_s, pg_s, pb_s, pa_s)
    sems = (sem0, sem1)
    ylo_f = (c * _HHALF).astype(jnp.float32)

    for ch in range(4):
        @pl.when(s == ch)
        def _zero():
            pltpu.sync_copy(zer_hbm, planes[ch])

    base = s * _CHUNK
    pltpu.sync_copy(flg_hbm.at[pl.ds(c * _NPAD + base, _CHUNK)], flg_v)

    cursor = jnp.int32(0)
    for k in range(_CHUNK // 16):
        gvec = k * 16 + lax.iota(jnp.int32, 16)
        m = flg_v[pl.ds(k * 16, 16)] > 0.0
        pos = cursor + plsc.cumsum(m.astype(jnp.int32)) - 1
        plsc.store_scatter(gidx_v, [pos], gvec, mask=m)
        cursor = cursor + plsc.all_reduce_population_count(m)[0]
    count = cursor
    gidx_v[pl.ds(count, 16)] = jnp.full((16,), _NPAD, jnp.int32) - base
    trip = (count + 2 * _GRP - 1) // (2 * _GRP)

    plsc.subcore_barrier()

    def _drain(b):
        pltpu.make_async_copy(zer_hbm.at[pl.ds(0, 4 * _GLANES)],
                              vals_v.at[b], sems[b]).wait()

    def _build_group(gvec, b):
        for gg in range(_GRP):
            g = gvec[b * _GRP + gg]
            prow = prmg_v[b * _GRP + gg, pl.ds(0, 16)]
            u = prow[0]
            v = prow[1]
            inv_a = prow[2]
            inv_b2 = prow[3] * 2.0
            inv_d = prow[4]
            r2 = prow[5]
            opac = prow[6]
            cr = prow[7]
            cg = prow[8]
            cb = prow[9]
            cxpf = prow[11]
            cypf = prow[12]
            cx1 = cxpf + (0.5 - u)
            cy1 = cypf + (0.5 - v)
            yoff = cxpf - ylo_f * float(_W)
            yhi_f = ylo_f + float(_HHALF - 1)
            for k in range(_KV):
                dx, dy, validc = _patch_consts(k)
                pxf = cxpf + dx
                pyf = cypf + dy
                fxr = dx + cx1
                fyr = dy + cy1
                fx2 = fxr * fxr
                fy2 = fyr * fyr
                power = -0.5 * (inv_a * fx2 + inv_b2 * (fxr * fyr)
                                + inv_d * fy2)
                gauss = jnp.exp(jnp.minimum(power, 0.0))
                alpha = jnp.minimum(opac * gauss, 0.99)
                msk = (pxf >= 0.0) & (pxf <= float(_W - 1)) \
                    & (pyf >= ylo_f) & (pyf <= yhi_f) \
                    & (fx2 + fy2 <= r2) & (alpha >= 1.0 / 255.0)
                if k == _KV - 1:
                    msk = msk & validc
                alpha = jnp.where(msk, alpha, 0.0)
                idxf = pyf * float(_W) + (dx + yoff)
                idx = jnp.clip(idxf, 0.0, float(_PIXH - 1)).astype(jnp.int32)
                idx_v[b * _KV + k, pl.ds(gg * 16, 16)] = idx
                off = k * 128 + gg * 16
                vals_v[b, pl.ds(0 * _GLANES + off, 16)] = alpha * cr
                vals_v[b, pl.ds(1 * _GLANES + off, 16)] = alpha * cg
                vals_v[b, pl.ds(2 * _GLANES + off, 16)] = alpha * cb
                vals_v[b, pl.ds(3 * _GLANES + off, 16)] = alpha

    def _issue_group(b):
        def _chunk(j, _):
            for ch in range(4):
                pltpu.async_copy(
                    vals_v.at[b, pl.ds(ch * _GLANES + j * 128, 128)],
                    planes[ch].at[idx_v.at[b * _KV + j]],
                    sems[b], add=True)
            return 0
        lax.fori_loop(0, _KV, _chunk, 0)

    def _main(i, _):
        gvec = gidx_v[pl.ds(i * 2 * _GRP, 16)]
        g16_v[...] = gvec + base
        pltpu.async_copy(prm_hbm.at[g16_v], prmg_v, gsem).wait()
        for b in range(2):
            @pl.when(i >= 1)
            def _w():
                _drain(b)
            _build_group(gvec, b)
            _issue_group(b)
        return 0

    lax.fori_loop(0, trip, _main, 0)

    @pl.when(trip >= 1)
    def _final_drain():
        _drain(0)
        _drain(1)

    plsc.subcore_barrier()

    p0 = s * _STRIPE
    nbufs = (n0_v, n1_v, n2_v, n3_v)
    for q in range(_STRIPE // _NCHUNK):
        off = p0 + q * _NCHUNK
        for ch in range(4):
            pltpu.sync_copy(planes[ch].at[pl.ds(off, _NCHUNK)], nbufs[ch])

        def _norm(k, _):
            sl = pl.ds(k * 16, 16)
            acc_a = n3_v[sl]
            scale = jnp.minimum(jnp.maximum(acc_a, 0.0), 1.0) \
                / jnp.maximum(acc_a, 1e-6)
            n0_v[sl] = n0_v[sl] * scale
            n1_v[sl] = n1_v[sl] * scale
            n2_v[sl] = n2_v[sl] * scale
            return 0

        lax.fori_loop(0, _NCHUNK // 16, _norm, 0)
        gb = c * _PIXH + off
        pltpu.sync_copy(n0_v, out_hbm.at[pl.ds(0 * _H * _W + gb, _NCHUNK)])
        pltpu.sync_copy(n1_v, out_hbm.at[pl.ds(1 * _H * _W + gb, _NCHUNK)])
        pltpu.sync_copy(n2_v, out_hbm.at[pl.ds(2 * _H * _W + gb, _NCHUNK)])


_sc_splat = functools.partial(
    pl.kernel,
    out_type=jax.ShapeDtypeStruct((3 * _H * _W,), jnp.float32),
    mesh=plsc.VectorSubcoreMesh(core_axis_name="c", subcore_axis_name="s"),
    compiler_params=pltpu.CompilerParams(needs_layout_passes=False),
    scratch_types=[
        pltpu.VMEM((_CHUNK,), jnp.float32),
        pltpu.VMEM((_CHUNK + 32,), jnp.int32),
        pltpu.VMEM((16,), jnp.int32),
        pltpu.VMEM((16, 128), jnp.float32),
        pltpu.VMEM((2, 4 * _GLANES), jnp.float32),
        pltpu.VMEM((2 * _KV, 128), jnp.int32),
        pltpu.VMEM((_NCHUNK,), jnp.float32),
        pltpu.VMEM((_NCHUNK,), jnp.float32),
        pltpu.VMEM((_NCHUNK,), jnp.float32),
        pltpu.VMEM((_NCHUNK,), jnp.float32),
        pltpu.VMEM_SHARED((_PIXH,), jnp.float32),
        pltpu.VMEM_SHARED((_PIXH,), jnp.float32),
        pltpu.VMEM_SHARED((_PIXH,), jnp.float32),
        pltpu.VMEM_SHARED((_PIXH,), jnp.float32),
        pltpu.SemaphoreType.DMA,
        pltpu.SemaphoreType.DMA,
        pltpu.SemaphoreType.DMA,
    ],
)(_sc_splat_body)


def kernel(positions, scales, rotations, colors, opacities):
    posT = positions.T
    sclT = scales.T
    rotT = rotations.T
    colT = colors.T
    opaT = opacities.reshape(1, _N)
    params = pl.pallas_call(
        _params_body,
        out_shape=jax.ShapeDtypeStruct((16, _N), jnp.float32),
    )(posT, sclT, rotT, colT, opaT)
    paramsP = jnp.pad(params, ((0, 0), (0, _NPAD - _N + 16)))
    paramsN = jnp.pad(paramsP.T, ((0, 0), (0, 112)))
    flags = jnp.concatenate([paramsP[10, :_NPAD], paramsP[15, :_NPAD]])
    zeros_h = jnp.zeros((_PIXH,), jnp.float32)
    img = _sc_splat(paramsN, flags, zeros_h)
    return jnp.moveaxis(img.reshape(3, _H, _W), 0, 2)


# speedup vs baseline: 1.1585x; 1.1585x over previous
import functools
import jax
import jax.numpy as jnp
from jax import lax
from jax.experimental import pallas as pl
from jax.experimental.pallas import tpu as pltpu
from jax.experimental.pallas import tpu_sc as plsc

_N = 20000
_H = 512
_W = 512
_FX = 600.0
_FY = 600.0
_CX = 256.0
_CY = 256.0
_NEAR = 0.01
_FAR = 100.0
_MAXR = 8

_NC = 2
_NS = 16
_NPAD = 20480
_CHUNK = _NPAD // _NS
_PATCH = (2 * _MAXR + 1) ** 2
_KV = 19
_GRP = 8
_GLANES = _GRP * _KV * 16
_HHALF = _H // _NC
_PIXH = _HHALF * _W
_STRIPE = _PIXH // _NS
_NCHUNK = 2048


def _params_body(pos_ref, scl_ref, rot_ref, col_ref, opa_ref, out_ref):
    x = pos_ref[0:1, :]
    y = pos_ref[1:2, :]
    z = pos_ref[2:3, :]
    depths = -z
    qw = rot_ref[0:1, :]
    qx = rot_ref[1:2, :]
    qy = rot_ref[2:3, :]
    qz = rot_ref[3:4, :]
    qn = jnp.maximum(jnp.sqrt(qw * qw + qx * qx + qy * qy + qz * qz), 1e-12)
    qw = qw / qn
    qx = qx / qn
    qy = qy / qn
    qz = qz / qn
    r00 = 1 - 2 * (qy * qy + qz * qz)
    r01 = 2 * (qx * qy - qw * qz)
    r02 = 2 * (qx * qz + qw * qy)
    r10 = 2 * (qx * qy + qw * qz)
    r11 = 1 - 2 * (qx * qx + qz * qz)
    r12 = 2 * (qy * qz - qw * qx)
    r20 = 2 * (qx * qz - qw * qy)
    r21 = 2 * (qy * qz + qw * qx)
    r22 = 1 - 2 * (qx * qx + qy * qy)
    s0 = scl_ref[0:1, :]
    s1 = scl_ref[1:2, :]
    s2 = scl_ref[2:3, :]
    v0 = s0 * s0
    v1 = s1 * s1
    v2 = s2 * s2
    c00 = r00 * r00 * v0 + r01 * r01 * v1 + r02 * r02 * v2
    c01 = r00 * r10 * v0 + r01 * r11 * v1 + r02 * r12 * v2
    c02 = r00 * r20 * v0 + r01 * r21 * v1 + r02 * r22 * v2
    c11 = r10 * r10 * v0 + r11 * r11 * v1 + r12 * r12 * v2
    c12 = r10 * r20 * v0 + r11 * r21 * v1 + r12 * r22 * v2
    c22 = r20 * r20 * v0 + r21 * r21 * v1 + r22 * r22 * v2
    z_safe = jnp.maximum(jnp.abs(z), 0.01) * jnp.sign(z + 1e-8)
    z2 = z_safe * z_safe
    j00 = _FX / -z_safe
    j02 = _FX * x / z2
    j11 = _FY / z_safe
    j12 = _FY * y / z2
    a = j00 * j00 * c00 + 2.0 * j00 * j02 * c02 + j02 * j02 * c22
    b = j00 * j11 * c01 + j00 * j12 * c02 + j02 * j11 * c12 + j02 * j12 * c22
    d = j11 * j11 * c11 + 2.0 * j11 * j12 * c12 + j12 * j12 * c22
    u = _FX * x / -z_safe + _CX
    v = _FY * -y / -z_safe + _CY
    trace = a + d
    det = jnp.maximum(a * d - b * b, 1e-6)
    disc = jnp.maximum(trace * trace - 4.0 * det, 0.0)
    lam = (trace + jnp.sqrt(disc)) * 0.5
    radii = jnp.minimum(3.0 * jnp.sqrt(jnp.maximum(lam, 1e-6)), float(_MAXR))
    vis = (depths > _NEAR) & (depths < _FAR)
    vis = vis & (u + radii > 0) & (u - radii < _W)
    vis = vis & (v + radii > 0) & (v - radii < _H)
    opac = opa_ref[0:1, :]
    vis = vis & (opac >= 1.0 / 255.0)
    aa = a + 0.3
    dd = d + 0.3
    det2 = jnp.maximum(aa * dd - b * b, 1e-6)
    inv_a = dd / det2
    inv_b = -b / det2
    inv_d = aa / det2
    cxpf = jnp.round(u)
    cypf = jnp.round(v)
    half0 = vis & (cypf + _MAXR >= 0) & (cypf - _MAXR <= _HHALF - 1)
    half1 = vis & (cypf + _MAXR >= _HHALF) & (cypf - _MAXR <= _H - 1)
    out_ref[0:1, :] = u
    out_ref[1:2, :] = v
    out_ref[2:3, :] = inv_a
    out_ref[3:4, :] = inv_b
    out_ref[4:5, :] = inv_d
    out_ref[5:6, :] = radii * radii
    out_ref[6:7, :] = opac
    out_ref[7:8, :] = col_ref[0:1, :]
    out_ref[8:9, :] = col_ref[1:2, :]
    out_ref[9:10, :] = col_ref[2:3, :]
    out_ref[10:11, :] = half0.astype(jnp.float32)
    out_ref[11:12, :] = cxpf
    out_ref[12:13, :] = cypf
    out_ref[13:14, :] = jnp.zeros_like(u)
    out_ref[14:15, :] = jnp.zeros_like(u)
    out_ref[15:16, :] = half1.astype(jnp.float32)


def _patch_consts(k):
    o = lax.iota(jnp.int32, 16) + (k * 16)
    dx = (o % 17 - _MAXR).astype(jnp.float32)
    dy = (o // 17 - _MAXR).astype(jnp.float32)
    valid = o < _PATCH
    return dx, dy, valid


def _sc_splat_body(prm_hbm, flg_hbm, zer_hbm, out_hbm,
                   flg_v, gidx_v, g16_v, prmg_v, vals_v, idx_v,
                   n0_v, n1_v, n2_v, n3_v,
                   pr_s, pg_s, pb_s, pa_s, sem0, sem1, gsem):
    c = lax.axis_index("c")
    s = lax.axis_index("s")
    planes = (pr_s, pg_s, pb_s, pa_s)
    sems = (sem0, sem1)
    ylo_f = (c * _HHALF).astype(jnp.float32)

    for ch in range(4):
        @pl.when(s == ch)
        def _zero():
            pltpu.sync_copy(zer_hbm, planes[ch])

    base = s * _CHUNK
    pltpu.sync_copy(flg_hbm.at[pl.ds(c * _NPAD + base, _CHUNK)], flg_v)

    cursor = jnp.int32(0)
    for k in range(_CHUNK // 16):
        gvec = k * 16 + lax.iota(jnp.int32, 16)
        m = flg_v[pl.ds(k * 16, 16)] > 0.0
        pos = cursor + plsc.cumsum(m.astype(jnp.int32)) - 1
        plsc.store_scatter(gidx_v, [pos], gvec, mask=m)
        cursor = cursor + plsc.all_reduce_population_count(m)[0]
    count = cursor
    gidx_v[pl.ds(count, 16)] = jnp.full((16,), _NPAD, jnp.int32) - base
    trip = (count + 2 * _GRP - 1) // (2 * _GRP)

    plsc.subcore_barrier()

    def _drain(b):
        pltpu.make_async_copy(zer_hbm.at[pl.ds(0, 4 * _GLANES)],
                              vals_v.at[b], sems[b]).wait()

    def _build_group(gvec, b):
        for gg in range(_GRP):
            g = gvec[b * _GRP + gg]
            prow = prmg_v[b * _GRP + gg, pl.ds(0, 16)]
            u = prow[0]
            v = prow[1]
            inv_a = prow[2]
            inv_b = prow[3]
            inv_d = prow[4]
            r2 = prow[5]
            opac = prow[6]
            cr = prow[7]
            cg = prow[8]
            cb = prow[9]
            cxpf = prow[11]
            cypf = prow[12]
            sbase = g * (_KV * 16)
            for k in range(_KV):
                dx, dy, validc = _patch_consts(k)
                pxf = cxpf + dx
                pyf = cypf + dy
                fxr = pxf + 0.5 - u
                fyr = pyf + 0.5 - v
                power = -0.5 * (inv_a * fxr * fxr + 2.0 * inv_b * fxr * fyr
                                + inv_d * fyr * fyr)
                gauss = jnp.exp(jnp.minimum(power, 0.0))
                alpha = jnp.minimum(opac * gauss, 0.99)
                msk = validc & (pxf >= 0.0) & (pxf <= float(_W - 1)) \
                    & (pyf >= ylo_f) & (pyf <= ylo_f + float(_HHALF - 1)) \
                    & (fxr * fxr + fyr * fyr <= r2)
                alpha = jnp.where(msk, alpha, 0.0)
                alpha = jnp.where(alpha < 1.0 / 255.0, 0.0, alpha)
                live = alpha > 0.0
                idxf = (pyf - ylo_f) * float(_W) + pxf
                idx = jnp.where(
                    live,
                    idxf.astype(jnp.int32),
                    (sbase + k * 16 + lax.iota(jnp.int32, 16))
                    & (_PIXH - 1))
                idx_v[b * _KV + k, pl.ds(gg * 16, 16)] = idx
                off = k * 128 + gg * 16
                vals_v[b, pl.ds(0 * _GLANES + off, 16)] = alpha * cr
                vals_v[b, pl.ds(1 * _GLANES + off, 16)] = alpha * cg
                vals_v[b, pl.ds(2 * _GLANES + off, 16)] = alpha * cb
                vals_v[b, pl.ds(3 * _GLANES + off, 16)] = alpha

    def _issue_group(b):
        def _chunk(j, _):
            for ch in range(4):
                pltpu.async_copy(
                    vals_v.at[b, pl.ds(ch * _GLANES + j * 128, 128)],
                    planes[ch].at[idx_v.at[b * _KV + j]],
                    sems[b], add=True)
            return 0
        lax.fori_loop(0, _KV, _chunk, 0)

    def _main(i, _):
        gvec = gidx_v[pl.ds(i * 2 * _GRP, 16)]
        g16_v[...] = gvec + base
        pltpu.async_copy(prm_hbm.at[g16_v], prmg_v, gsem).wait()
        for b in range(2):
            @pl.when(i >= 1)
            def _w():
                _drain(b)
            _build_group(gvec, b)
            _issue_group(b)
        return 0

    lax.fori_loop(0, trip, _main, 0)

    @pl.when(trip >= 1)
    def _final_drain():
        _drain(0)
        _drain(1)

    plsc.subcore_barrier()

    p0 = s * _STRIPE
    nbufs = (n0_v, n1_v, n2_v, n3_v)
    for q in range(_STRIPE // _NCHUNK):
        off = p0 + q * _NCHUNK
        for ch in range(4):
            pltpu.sync_copy(planes[ch].at[pl.ds(off, _NCHUNK)], nbufs[ch])

        def _norm(k, _):
            sl = pl.ds(k * 16, 16)
            acc_a = n3_v[sl]
            scale = jnp.minimum(jnp.maximum(acc_a, 0.0), 1.0) \
                / jnp.maximum(acc_a, 1e-6)
            n0_v[sl] = n0_v[sl] * scale
            n1_v[sl] = n1_v[sl] * scale
            n2_v[sl] = n2_v[sl] * scale
            return 0

        lax.fori_loop(0, _NCHUNK // 16, _norm, 0)
        gb = c * _PIXH + off
        pltpu.sync_copy(n0_v, out_hbm.at[pl.ds(0 * _H * _W + gb, _NCHUNK)])
        pltpu.sync_copy(n1_v, out_hbm.at[pl.ds(1 * _H * _W + gb, _NCHUNK)])
        pltpu.sync_copy(n2_v, out_hbm.at[pl.ds(2 * _H * _W + gb, _NCHUNK)])


_sc_splat = functools.partial(
    pl.kernel,
    out_type=jax.ShapeDtypeStruct((3 * _H * _W,), jnp.float32),
    mesh=plsc.VectorSubcoreMesh(core_axis_name="c", subcore_axis_name="s"),
    compiler_params=pltpu.CompilerParams(needs_layout_passes=False),
    scratch_types=[
        pltpu.VMEM((_CHUNK,), jnp.float32),
        pltpu.VMEM((_CHUNK + 32,), jnp.int32),
        pltpu.VMEM((16,), jnp.int32),
        pltpu.VMEM((16, 128), jnp.float32),
        pltpu.VMEM((2, 4 * _GLANES), jnp.float32),
        pltpu.VMEM((2 * _KV, 128), jnp.int32),
        pltpu.VMEM((_NCHUNK,), jnp.float32),
        pltpu.VMEM((_NCHUNK,), jnp.float32),
        pltpu.VMEM((_NCHUNK,), jnp.float32),
        pltpu.VMEM((_NCHUNK,), jnp.float32),
        pltpu.VMEM_SHARED((_PIXH,), jnp.float32),
        pltpu.VMEM_SHARED((_PIXH,), jnp.float32),
        pltpu.VMEM_SHARED((_PIXH,), jnp.float32),
        pltpu.VMEM_SHARED((_PIXH,), jnp.float32),
        pltpu.SemaphoreType.DMA,
        pltpu.SemaphoreType.DMA,
        pltpu.SemaphoreType.DMA,
    ],
)(_sc_splat_body)


def kernel(positions, scales, rotations, colors, opacities):
    posT = positions.T
    sclT = scales.T
    rotT = rotations.T
    colT = colors.T
    opaT = opacities.reshape(1, _N)
    params = pl.pallas_call(
        _params_body,
        out_shape=jax.ShapeDtypeStruct((16, _N), jnp.float32),
    )(posT, sclT, rotT, colT, opaT)
    paramsP = jnp.pad(params, ((0, 0), (0, _NPAD - _N + 16)))
    paramsN = jnp.pad(paramsP.T, ((0, 0), (0, 112)))
    flags = jnp.concatenate([paramsP[10, :_NPAD], paramsP[15, :_NPAD]])
    zeros_h = jnp.zeros((_PIXH,), jnp.float32)
    img = _sc_splat(paramsN, flags, zeros_h)
    return jnp.moveaxis(img.reshape(3, _H, _W), 0, 2)
